# baseline (speedup 1.0000x reference)
import functools

import jax
import jax.numpy as jnp
from jax import lax
from jax.experimental import pallas as pl
from jax.experimental.pallas import tpu as pltpu
from jax.experimental.pallas import tpu_sc as plsc

_VOCAB = 1000000
_SPLIT = 500224
_EMB = 64
_TAG = 64
_BATCH = 1024
_SEQ = 200

_NTOK = _BATCH * _SEQ
_NW = 32
_PER_W = _NTOK // _NW
_CH = 128
_NCH = _PER_W // _CH

_P1_BU = 512
_P1_GRID = _SPLIT // _P1_BU


def _p1_body(x1_ref, x2_ref, w_ref, b_ref, o_ref):
    w = w_ref[...]
    b = b_ref[...]
    dn = (((0,), (0,)), ((), ()))
    o_ref[:, 0:64] = (
        lax.dot_general(x1_ref[...], w, dn, preferred_element_type=jnp.float32)
        + b
    )
    o_ref[:, 64:128] = (
        lax.dot_general(x2_ref[...], w, dn, preferred_element_type=jnp.float32)
        + b
    )


def _project_table(table_t, W, b):
    return pl.pallas_call(
        _p1_body,
        grid=(_P1_GRID,),
        in_specs=[
            pl.BlockSpec((_EMB, _P1_BU), lambda i: (0, i)),
            pl.BlockSpec((_EMB, _P1_BU), lambda i: (0, i + _P1_GRID)),
            pl.BlockSpec((_EMB, _TAG), lambda i: (0, 0)),
            pl.BlockSpec((1, _TAG), lambda i: (0, 0)),
        ],
        out_specs=pl.BlockSpec((_P1_BU, 2 * _TAG), lambda i: (i, 0)),
        out_shape=jax.ShapeDtypeStruct((_SPLIT, 2 * _TAG), jnp.float32),
    )(table_t, table_t, W, b.reshape(1, _TAG))


def _make_sc_gather():
    mesh = plsc.VectorSubcoreMesh(core_axis_name="c", subcore_axis_name="s")

    @functools.partial(
        pl.kernel,
        mesh=mesh,
        out_type=jax.ShapeDtypeStruct((_NTOK, _EMB), jnp.float32),
        scratch_types=[
            pltpu.VMEM((_NCH, _CH), jnp.int32),
            pltpu.VMEM((_CH, _EMB), jnp.float32),
            pltpu.SemaphoreType.DMA,
        ],
        compiler_params=pltpu.CompilerParams(use_tc_tiling_on_sc=False),
    )
    def gather_kernel(table_hbm, idx_hbm, out_hbm, idx_v, rows_v, sem):
        wid = lax.axis_index("s") * 2 + lax.axis_index("c")
        base = wid * _PER_W
        pltpu.sync_copy(idx_hbm.at[wid], idx_v)

        def body(c, carry):
            pltpu.async_copy(table_hbm.at[idx_v.at[c]], rows_v, sem).wait()
            pltpu.sync_copy(rows_v, out_hbm.at[pl.ds(base + c * _CH, _CH)])
            return carry

        lax.fori_loop(0, _NCH, body, 0)

    return gather_kernel


_sc_gather = _make_sc_gather()


def _p3_body(x_ref, o_ref):
    x = x_ref[0]
    ii = lax.broadcasted_iota(jnp.int32, (_TAG, _TAG), 0)
    jj = lax.broadcasted_iota(jnp.int32, (_TAG, _TAG), 1)
    eye = (ii == jj).astype(jnp.float32)
    dn = (((1,), (1,)), ((), ()))
    o_ref[0, :, 0:512] = lax.dot_general(
        eye, x[:, 0:64], dn, preferred_element_type=jnp.float32)
    o_ref[0, :, 512:1024] = lax.dot_general(
        eye, x[:, 64:128], dn, preferred_element_type=jnp.float32)


def _final_transpose(g3):
    return pl.pallas_call(
        _p3_body,
        grid=(_SEQ,),
        in_specs=[pl.BlockSpec((1, 512, 128), lambda i: (i, 0, 0))],
        out_specs=pl.BlockSpec((1, _TAG, _BATCH), lambda i: (i, 0, 0)),
        out_shape=jax.ShapeDtypeStruct((_SEQ, _TAG, _BATCH), jnp.float32),
    )(g3)


def kernel(emb_table, W, b, batch_w, batch_x, batch_w_lengths, batch_x_lengths):
    table_t = emb_table.T
    tp = _project_table(table_t, W, b)
    tp_lin = tp.reshape(2 * _SPLIT, _EMB)

    bx_t = batch_x.T.astype(jnp.int32)
    v = bx_t.reshape(_SEQ, 2, 512).swapaxes(1, 2)
    idxp = jnp.where(v < _SPLIT, 2 * v, 2 * (v - _SPLIT) + 1)
    idx3 = idxp.reshape(_NW, _NCH, _CH)

    g = _sc_gather(tp_lin, idx3)
    g3 = g.reshape(_SEQ, 512, 128)
    out_t = _final_transpose(g3)
    return out_t.transpose(2, 0, 1)


# speedup vs baseline: 1.0351x; 1.0351x over previous
import functools

import jax
import jax.numpy as jnp
from jax import lax
from jax.experimental import pallas as pl
from jax.experimental.pallas import tpu as pltpu
from jax.experimental.pallas import tpu_sc as plsc

_VOCAB = 1000000
_SPLIT = 507904
_EMB = 64
_TAG = 64
_BATCH = 1024
_SEQ = 200

_NTOK = _BATCH * _SEQ
_NW = 32
_PER_W = _NTOK // _NW
_CH = 128
_NCH = _PER_W // _CH

_P1_BU = 16384
_P1_GRID = _SPLIT // _P1_BU
_PHASE_CUT = 0


def _p1_body(x1_ref, x2_ref, w_ref, b_ref, o_ref):
    w = w_ref[...]
    b = b_ref[...]
    dn = (((0,), (0,)), ((), ()))
    o_ref[:, 0:64] = (
        lax.dot_general(x1_ref[...], w, dn, preferred_element_type=jnp.float32)
        + b
    ).astype(jnp.bfloat16)
    o_ref[:, 64:128] = (
        lax.dot_general(x2_ref[...], w, dn, preferred_element_type=jnp.float32)
        + b
    ).astype(jnp.bfloat16)


def _project_table(table_t, W, b):
    return pl.pallas_call(
        _p1_body,
        grid=(_P1_GRID,),
        in_specs=[
            pl.BlockSpec((_EMB, _P1_BU), lambda i: (0, i)),
            pl.BlockSpec(
                (_EMB, _P1_BU),
                lambda i: (0, jnp.minimum(i + _P1_GRID, _VOCAB // _P1_BU)),
            ),
            pl.BlockSpec((_EMB, _TAG), lambda i: (0, 0)),
            pl.BlockSpec((1, _TAG), lambda i: (0, 0)),
        ],
        out_specs=pl.BlockSpec((_P1_BU, 2 * _TAG), lambda i: (i, 0)),
        out_shape=jax.ShapeDtypeStruct((_SPLIT, 2 * _TAG), jnp.bfloat16),
    )(table_t, table_t, W, b.reshape(1, _TAG))


def _make_sc_gather():
    mesh = plsc.VectorSubcoreMesh(core_axis_name="c", subcore_axis_name="s")

    @functools.partial(
        pl.kernel,
        mesh=mesh,
        out_type=jax.ShapeDtypeStruct((_NTOK, _EMB), jnp.bfloat16),
        scratch_types=[
            pltpu.VMEM((_NCH, _CH), jnp.int32),
            pltpu.VMEM((_CH, _EMB), jnp.bfloat16),
            pltpu.SemaphoreType.DMA,
        ],
        compiler_params=pltpu.CompilerParams(use_tc_tiling_on_sc=False),
    )
    def gather_kernel(table_hbm, idx_hbm, out_hbm, idx_v, rows_v, sem):
        wid = lax.axis_index("s") * 2 + lax.axis_index("c")
        base = wid * _PER_W
        pltpu.sync_copy(idx_hbm.at[wid], idx_v)

        def body(c, carry):
            pltpu.async_copy(table_hbm.at[idx_v.at[c]], rows_v, sem).wait()
            pltpu.sync_copy(rows_v, out_hbm.at[pl.ds(base + c * _CH, _CH)])
            return carry

        lax.fori_loop(0, _NCH, body, 0)

    return gather_kernel


_sc_gather = _make_sc_gather()


def _p3_body(x_ref, o_ref):
    x = x_ref[0].astype(jnp.float32)
    ii = lax.broadcasted_iota(jnp.int32, (_TAG, _TAG), 0)
    jj = lax.broadcasted_iota(jnp.int32, (_TAG, _TAG), 1)
    eye = (ii == jj).astype(jnp.float32)
    dn = (((1,), (1,)), ((), ()))
    o_ref[0, :, 0:512] = lax.dot_general(
        eye, x[:, 0:64], dn, preferred_element_type=jnp.float32)
    o_ref[0, :, 512:1024] = lax.dot_general(
        eye, x[:, 64:128], dn, preferred_element_type=jnp.float32)


def _final_transpose(g3):
    return pl.pallas_call(
        _p3_body,
        grid=(_SEQ,),
        in_specs=[pl.BlockSpec((1, 512, 128), lambda i: (i, 0, 0))],
        out_specs=pl.BlockSpec((1, _TAG, _BATCH), lambda i: (i, 0, 0)),
        out_shape=jax.ShapeDtypeStruct((_SEQ, _TAG, _BATCH), jnp.float32),
    )(g3)


def kernel(emb_table, W, b, batch_w, batch_x, batch_w_lengths, batch_x_lengths):
    table_t = emb_table.T
    tp = _project_table(table_t, W, b)
    tp_lin = tp.reshape(2 * _SPLIT, _EMB)

    bx_t = batch_x.T.astype(jnp.int32)
    v = bx_t.reshape(_SEQ, 2, 512).swapaxes(1, 2)
    idxp = jnp.where(v < _SPLIT, 2 * v, 2 * (v - _SPLIT) + 1)
    idx3 = idxp.reshape(_NW, _NCH, _CH)

    if _PHASE_CUT == 1:
        return tp
    g = _sc_gather(tp_lin, idx3)
    if _PHASE_CUT == 2:
        return g
    g3 = g.reshape(_SEQ, 512, 128)
    out_t = _final_transpose(g3)
    return out_t.transpose(2, 0, 1)


# speedup vs baseline: 2.0135x; 1.9453x over previous
import functools

import jax
import jax.numpy as jnp
from jax import lax
from jax.experimental import pallas as pl
from jax.experimental.pallas import tpu as pltpu
from jax.experimental.pallas import tpu_sc as plsc

_VOCAB = 1000000
_SPLIT = 507904
_EMB = 64
_TAG = 64
_BATCH = 1024
_SEQ = 200

_NTOK = _BATCH * _SEQ
_NW = 32
_PER_W = _NTOK // _NW
_CH = 128
_NCH = _PER_W // _CH

_P1_BU = 16384
_P1_GRID = _SPLIT // _P1_BU
_PHASE_CUT = 0


def _p1_body(x1_ref, x2_ref, w_ref, b_ref, o_ref):
    w = w_ref[...]
    b = b_ref[...]
    dn = (((0,), (0,)), ((), ()))
    o_ref[:, 0:64] = (
        lax.dot_general(x1_ref[...], w, dn, preferred_element_type=jnp.float32)
        + b
    )
    o_ref[:, 64:128] = (
        lax.dot_general(x2_ref[...], w, dn, preferred_element_type=jnp.float32)
        + b
    )


def _project_table(table_t, W, b):
    return pl.pallas_call(
        _p1_body,
        grid=(_P1_GRID,),
        in_specs=[
            pl.BlockSpec((_EMB, _P1_BU), lambda i: (0, i)),
            pl.BlockSpec(
                (_EMB, _P1_BU),
                lambda i: (0, jnp.minimum(i + _P1_GRID, _VOCAB // _P1_BU)),
            ),
            pl.BlockSpec((_EMB, _TAG), lambda i: (0, 0)),
            pl.BlockSpec((1, _TAG), lambda i: (0, 0)),
        ],
        out_specs=pl.BlockSpec((_P1_BU, 2 * _TAG), lambda i: (i, 0)),
        out_shape=jax.ShapeDtypeStruct((_SPLIT, 2 * _TAG), jnp.float32),
    )(table_t, table_t, W, b.reshape(1, _TAG))


def _make_sc_gather():
    mesh = plsc.VectorSubcoreMesh(core_axis_name="c", subcore_axis_name="s")

    @functools.partial(
        pl.kernel,
        mesh=mesh,
        out_type=jax.ShapeDtypeStruct((_NTOK, _EMB), jnp.float32),
        scratch_types=[
            pltpu.VMEM((_NCH, _CH), jnp.int32),
            pltpu.VMEM((_CH, _EMB), jnp.float32),
            pltpu.SemaphoreType.DMA,
        ],
        compiler_params=pltpu.CompilerParams(use_tc_tiling_on_sc=False),
    )
    def gather_kernel(table_hbm, idx_hbm, out_hbm, idx_v, rows_v, sem):
        wid = lax.axis_index("s") * 2 + lax.axis_index("c")
        base = wid * _PER_W
        pltpu.sync_copy(idx_hbm.at[wid], idx_v)

        def body(c, carry):
            pltpu.async_copy(table_hbm.at[idx_v.at[c]], rows_v, sem).wait()
            pltpu.sync_copy(rows_v, out_hbm.at[pl.ds(base + c * _CH, _CH)])
            return carry

        lax.fori_loop(0, _NCH, body, 0)

    return gather_kernel


_sc_gather = _make_sc_gather()


def _p3_body(x_ref, o_ref):
    x = x_ref[0]
    ii = lax.broadcasted_iota(jnp.int32, (_TAG, _TAG), 0)
    jj = lax.broadcasted_iota(jnp.int32, (_TAG, _TAG), 1)
    eye = (ii == jj).astype(jnp.float32)
    dn = (((1,), (1,)), ((), ()))
    o_ref[0, :, 0:512] = lax.dot_general(
        eye, x[:, 0:64], dn, preferred_element_type=jnp.float32)
    o_ref[0, :, 512:1024] = lax.dot_general(
        eye, x[:, 64:128], dn, preferred_element_type=jnp.float32)


def _final_transpose(g3):
    return pl.pallas_call(
        _p3_body,
        grid=(_SEQ,),
        in_specs=[pl.BlockSpec((1, 512, 128), lambda i: (i, 0, 0))],
        out_specs=pl.BlockSpec((1, _TAG, _BATCH), lambda i: (i, 0, 0)),
        out_shape=jax.ShapeDtypeStruct((_SEQ, _TAG, _BATCH), jnp.float32),
    )(g3)


def kernel(emb_table, W, b, batch_w, batch_x, batch_w_lengths, batch_x_lengths):
    table_t = emb_table.T
    tp = _project_table(table_t, W, b)
    tp_lin = tp.reshape(2 * _SPLIT, _EMB)

    bx_t = batch_x.T.astype(jnp.int32)
    v = bx_t.reshape(_SEQ, 2, 512).swapaxes(1, 2)
    idxp = jnp.where(v < _SPLIT, 2 * v, 2 * (v - _SPLIT) + 1)
    idx3 = idxp.reshape(_NW, _NCH, _CH)

    if _PHASE_CUT == 1:
        return tp
    g = _sc_gather(tp_lin, idx3)
    if _PHASE_CUT == 2:
        return g
    g3 = g.reshape(_SEQ, 512, 128)
    out_t = _final_transpose(g3)
    return out_t.transpose(2, 0, 1)


# speedup vs baseline: 2.1515x; 1.0685x over previous
import functools

import jax
import jax.numpy as jnp
from jax import lax
from jax.experimental import pallas as pl
from jax.experimental.pallas import tpu as pltpu
from jax.experimental.pallas import tpu_sc as plsc

_VOCAB = 1000000
_SPLIT = 507904
_EMB = 64
_TAG = 64
_BATCH = 1024
_SEQ = 200

_NTOK = _BATCH * _SEQ
_NW = 32
_PER_W = _NTOK // _NW
_CH = 128
_NCH = _PER_W // _CH

_P1_BU = 16384
_P1_GRID = _SPLIT // _P1_BU
_PHASE_CUT = 0


def _p1_body(x1_ref, x2_ref, w_ref, b_ref, o_ref):
    w = w_ref[...]
    b = b_ref[...]
    dn = (((0,), (0,)), ((), ()))
    o_ref[:, 0:64] = (
        lax.dot_general(x1_ref[...], w, dn, preferred_element_type=jnp.float32)
        + b
    )
    o_ref[:, 64:128] = (
        lax.dot_general(x2_ref[...], w, dn, preferred_element_type=jnp.float32)
        + b
    )


def _project_table(table_t, W, b):
    return pl.pallas_call(
        _p1_body,
        grid=(_P1_GRID,),
        in_specs=[
            pl.BlockSpec((_EMB, _P1_BU), lambda i: (0, i)),
            pl.BlockSpec(
                (_EMB, _P1_BU),
                lambda i: (0, jnp.minimum(i + _P1_GRID, _VOCAB // _P1_BU)),
            ),
            pl.BlockSpec((_EMB, _TAG), lambda i: (0, 0)),
            pl.BlockSpec((1, _TAG), lambda i: (0, 0)),
        ],
        out_specs=pl.BlockSpec((_P1_BU, 2 * _TAG), lambda i: (i, 0)),
        out_shape=jax.ShapeDtypeStruct((_SPLIT, 2 * _TAG), jnp.float32),
    )(table_t, table_t, W, b.reshape(1, _TAG))


def _make_sc_gather():
    mesh = plsc.VectorSubcoreMesh(core_axis_name="c", subcore_axis_name="s")

    nsub = 5
    sup_rows = nsub * _CH
    nsup = _PER_W // sup_rows

    @functools.partial(
        pl.kernel,
        mesh=mesh,
        out_type=jax.ShapeDtypeStruct((_NTOK, _EMB), jnp.float32),
        scratch_types=[
            pltpu.VMEM((_NCH, _CH), jnp.int32),
            pltpu.VMEM((sup_rows, _EMB), jnp.float32),
            pltpu.SemaphoreType.DMA,
            pltpu.SemaphoreType.DMA,
        ],
        compiler_params=pltpu.CompilerParams(use_tc_tiling_on_sc=False),
    )
    def gather_kernel(table_hbm, idx_hbm, out_hbm, idx_v, rows_v, gsem, wsem):
        wid = lax.axis_index("s") * 2 + lax.axis_index("c")
        base = wid * _PER_W
        pltpu.sync_copy(idx_hbm.at[wid], idx_v)

        def body(g, carry):
            handles = []
            for j in range(nsub):
                h = pltpu.async_copy(
                    table_hbm.at[idx_v.at[g * nsub + j]],
                    rows_v.at[pl.ds(j * _CH, _CH)],
                    gsem,
                )
                handles.append(h)
            for h in handles:
                h.wait()
            pltpu.async_copy(
                rows_v, out_hbm.at[pl.ds(base + g * sup_rows, sup_rows)], wsem
            ).wait()
            return carry

        lax.fori_loop(0, nsup, body, 0)

    return gather_kernel


_sc_gather = _make_sc_gather()


def _p3_body(x_ref, o_ref):
    x = x_ref[0]
    ii = lax.broadcasted_iota(jnp.int32, (_TAG, _TAG), 0)
    jj = lax.broadcasted_iota(jnp.int32, (_TAG, _TAG), 1)
    eye = (ii == jj).astype(jnp.float32)
    dn = (((1,), (1,)), ((), ()))
    o_ref[0, :, 0:512] = lax.dot_general(
        eye, x[:, 0:64], dn, preferred_element_type=jnp.float32)
    o_ref[0, :, 512:1024] = lax.dot_general(
        eye, x[:, 64:128], dn, preferred_element_type=jnp.float32)


def _final_transpose(g3):
    return pl.pallas_call(
        _p3_body,
        grid=(_SEQ,),
        in_specs=[pl.BlockSpec((1, 512, 128), lambda i: (i, 0, 0))],
        out_specs=pl.BlockSpec((1, _TAG, _BATCH), lambda i: (i, 0, 0)),
        out_shape=jax.ShapeDtypeStruct((_SEQ, _TAG, _BATCH), jnp.float32),
    )(g3)


def kernel(emb_table, W, b, batch_w, batch_x, batch_w_lengths, batch_x_lengths):
    table_t = emb_table.T
    tp = _project_table(table_t, W, b)
    tp_lin = tp.reshape(2 * _SPLIT, _EMB)

    bx_t = batch_x.T.astype(jnp.int32)
    v = bx_t.reshape(_SEQ, 2, 512).swapaxes(1, 2)
    idxp = jnp.where(v < _SPLIT, 2 * v, 2 * (v - _SPLIT) + 1)
    idx3 = idxp.reshape(_NW, _NCH, _CH)

    if _PHASE_CUT == 1:
        return tp
    g = _sc_gather(tp_lin, idx3)
    if _PHASE_CUT == 2:
        return g
    g3 = g.reshape(_SEQ, 512, 128)
    out_t = _final_transpose(g3)
    return out_t.transpose(2, 0, 1)


# speedup vs baseline: 2.6194x; 1.2175x over previous
import functools

import jax
import jax.numpy as jnp
from jax import lax
from jax.experimental import pallas as pl
from jax.experimental.pallas import tpu as pltpu
from jax.experimental.pallas import tpu_sc as plsc

_VOCAB = 1000000
_SPLIT = 507904
_EMB = 64
_TAG = 64
_BATCH = 1024
_SEQ = 200

_NTOK = _BATCH * _SEQ
_NW = 32
_PER_W = _NTOK // _NW
_CH = 128
_NCH = _PER_W // _CH

_P1_BU = 16384
_P1_GRID = _SPLIT // _P1_BU
_PHASE_CUT = 0


def _p1_body(x1_ref, x2_ref, w_ref, b_ref, o_ref):
    w = w_ref[...]
    b = b_ref[...]
    dn = (((0,), (0,)), ((), ()))
    o_ref[:, 0:64] = (
        lax.dot_general(x1_ref[...], w, dn, preferred_element_type=jnp.float32)
        + b
    )
    o_ref[:, 64:128] = (
        lax.dot_general(x2_ref[...], w, dn, preferred_element_type=jnp.float32)
        + b
    )


def _project_table(table_t, W, b):
    return pl.pallas_call(
        _p1_body,
        grid=(_P1_GRID,),
        in_specs=[
            pl.BlockSpec((_EMB, _P1_BU), lambda i: (0, i)),
            pl.BlockSpec(
                (_EMB, _P1_BU),
                lambda i: (0, jnp.minimum(i + _P1_GRID, _VOCAB // _P1_BU)),
            ),
            pl.BlockSpec((_EMB, _TAG), lambda i: (0, 0)),
            pl.BlockSpec((1, _TAG), lambda i: (0, 0)),
        ],
        out_specs=pl.BlockSpec((_P1_BU, 2 * _TAG), lambda i: (i, 0)),
        out_shape=jax.ShapeDtypeStruct((_SPLIT, 2 * _TAG), jnp.float32),
    )(table_t, table_t, W, b.reshape(1, _TAG))


def _make_sc_gather():
    mesh = plsc.VectorSubcoreMesh(core_axis_name="c", subcore_axis_name="s")

    nsub = 5
    sup_rows = nsub * _CH
    nsup = _PER_W // sup_rows

    @functools.partial(
        pl.kernel,
        mesh=mesh,
        out_type=jax.ShapeDtypeStruct((_NTOK, _EMB), jnp.float32),
        scratch_types=[
            pltpu.VMEM((_NCH, _CH), jnp.int32),
            pltpu.VMEM((sup_rows, _EMB), jnp.float32),
            pltpu.SemaphoreType.DMA,
            pltpu.SemaphoreType.DMA,
        ],
        compiler_params=pltpu.CompilerParams(use_tc_tiling_on_sc=False),
    )
    def gather_kernel(table_hbm, idx_hbm, out_hbm, idx_v, rows_v, gsem, wsem):
        wid = lax.axis_index("s") * 2 + lax.axis_index("c")
        base = wid * _PER_W
        pltpu.sync_copy(idx_hbm.at[wid], idx_v)

        def body(g, carry):
            handles = []
            for j in range(nsub):
                h = pltpu.async_copy(
                    table_hbm.at[idx_v.at[g * nsub + j]],
                    rows_v.at[pl.ds(j * _CH, _CH)],
                    gsem,
                )
                handles.append(h)
            for h in handles:
                h.wait()
            pltpu.async_copy(
                rows_v, out_hbm.at[pl.ds(base + g * sup_rows, sup_rows)], wsem
            ).wait()
            return carry

        lax.fori_loop(0, nsup, body, 0)

    return gather_kernel


_sc_gather = _make_sc_gather()


_P3_SB = 8


def _p3_body(x_ref, o_ref):
    for si in range(_P3_SB):
        x = x_ref[si]
        o_ref[si, :, 0:512] = x[:, 0:64].T
        o_ref[si, :, 512:1024] = x[:, 64:128].T


def _final_transpose(g3):
    return pl.pallas_call(
        _p3_body,
        grid=(_SEQ // _P3_SB,),
        in_specs=[pl.BlockSpec((_P3_SB, 512, 128), lambda i: (i, 0, 0))],
        out_specs=pl.BlockSpec((_P3_SB, _TAG, _BATCH), lambda i: (i, 0, 0)),
        out_shape=jax.ShapeDtypeStruct((_SEQ, _TAG, _BATCH), jnp.float32),
    )(g3)


def kernel(emb_table, W, b, batch_w, batch_x, batch_w_lengths, batch_x_lengths):
    table_t = emb_table.T
    tp = _project_table(table_t, W, b)
    tp_lin = tp.reshape(2 * _SPLIT, _EMB)

    bx_t = batch_x.T.astype(jnp.int32)
    v = bx_t.reshape(_SEQ, 2, 512).swapaxes(1, 2)
    idxp = jnp.where(v < _SPLIT, 2 * v, 2 * (v - _SPLIT) + 1)
    idx3 = idxp.reshape(_NW, _NCH, _CH)

    if _PHASE_CUT == 1:
        return tp
    g = _sc_gather(tp_lin, idx3)
    if _PHASE_CUT == 2:
        return g
    g3 = g.reshape(_SEQ, 512, 128)
    out_t = _final_transpose(g3)
    return out_t.transpose(2, 0, 1)


# speedup vs baseline: 3.0630x; 1.1694x over previous
import functools

import jax
import jax.numpy as jnp
from jax import lax
from jax.experimental import pallas as pl
from jax.experimental.pallas import tpu as pltpu
from jax.experimental.pallas import tpu_sc as plsc

_VOCAB = 1000000
_SPLIT = 507904
_EMB = 64
_TAG = 64
_BATCH = 1024
_SEQ = 200

_NTOK = _BATCH * _SEQ
_NW = 32
_PER_W = _NTOK // _NW
_CH = 128
_NCH = _PER_W // _CH

_P1_BU = 16384
_P1_GRID = _SPLIT // _P1_BU
_PHASE_CUT = 0


def _p1_body(x1_ref, x2_ref, w_ref, b_ref, o_ref):
    w = w_ref[...]
    b = b_ref[...]
    dn = (((0,), (0,)), ((), ()))
    o_ref[:, 0:64] = (
        lax.dot_general(x1_ref[...], w, dn, preferred_element_type=jnp.float32)
        + b
    )
    o_ref[:, 64:128] = (
        lax.dot_general(x2_ref[...], w, dn, preferred_element_type=jnp.float32)
        + b
    )


def _project_table(table_t, W, b):
    return pl.pallas_call(
        _p1_body,
        grid=(_P1_GRID,),
        in_specs=[
            pl.BlockSpec((_EMB, _P1_BU), lambda i: (0, i)),
            pl.BlockSpec(
                (_EMB, _P1_BU),
                lambda i: (0, jnp.minimum(i + _P1_GRID, _VOCAB // _P1_BU)),
            ),
            pl.BlockSpec((_EMB, _TAG), lambda i: (0, 0)),
            pl.BlockSpec((1, _TAG), lambda i: (0, 0)),
        ],
        out_specs=pl.BlockSpec((_P1_BU, 2 * _TAG), lambda i: (i, 0)),
        out_shape=jax.ShapeDtypeStruct((_SPLIT, 2 * _TAG), jnp.float32),
    )(table_t, table_t, W, b.reshape(1, _TAG))


def _make_sc_gather():
    mesh = plsc.VectorSubcoreMesh(core_axis_name="c", subcore_axis_name="s")

    nsub = 5
    sup_rows = nsub * _CH
    nsup = _PER_W // sup_rows

    @functools.partial(
        pl.kernel,
        mesh=mesh,
        out_type=jax.ShapeDtypeStruct((_NTOK // 2, 2 * _EMB), jnp.float32),
        scratch_types=[
            pltpu.VMEM((_NCH, _CH), jnp.int32),
            pltpu.VMEM((sup_rows, _EMB), jnp.float32),
            pltpu.SemaphoreType.DMA,
            pltpu.SemaphoreType.DMA,
        ],
        compiler_params=pltpu.CompilerParams(use_tc_tiling_on_sc=False),
    )
    def gather_kernel(table_hbm, idx_hbm, out_hbm, idx_v, rows_v, gsem, wsem):
        wid = lax.axis_index("s") * 2 + lax.axis_index("c")
        rbase = wid * (_PER_W // 2)
        pltpu.sync_copy(idx_hbm.at[wid], idx_v)

        def body(g, carry):
            handles = []
            for j in range(nsub):
                h = pltpu.async_copy(
                    table_hbm.at[idx_v.at[g * nsub + j]],
                    rows_v.at[pl.ds(j * _CH, _CH)],
                    gsem,
                )
                handles.append(h)
            for h in handles:
                h.wait()
            ws = []
            for j in range(nsub):
                r0 = rbase + g * (sup_rows // 2) + j * 64
                ws.append(pltpu.async_copy(
                    rows_v.at[pl.ds(j * _CH, 64)],
                    out_hbm.at[pl.ds(r0, 64), pl.ds(0, 64)],
                    wsem,
                ))
                ws.append(pltpu.async_copy(
                    rows_v.at[pl.ds(j * _CH + 64, 64)],
                    out_hbm.at[pl.ds(r0, 64), pl.ds(64, 64)],
                    wsem,
                ))
            for h in ws:
                h.wait()
            return carry

        lax.fori_loop(0, nsup, body, 0)

    return gather_kernel


_sc_gather = _make_sc_gather()


_P3_SB = 8


def _p3_body(x_ref, o_ref):
    for si in range(_P3_SB):
        x = x_ref[si]
        o_ref[si, :, 0:512] = x[:, 0:64].T
        o_ref[si, :, 512:1024] = x[:, 64:128].T


def _final_transpose(g3):
    return pl.pallas_call(
        _p3_body,
        grid=(_SEQ // _P3_SB,),
        in_specs=[pl.BlockSpec((_P3_SB, 512, 128), lambda i: (i, 0, 0))],
        out_specs=pl.BlockSpec((_P3_SB, _TAG, _BATCH), lambda i: (i, 0, 0)),
        out_shape=jax.ShapeDtypeStruct((_SEQ, _TAG, _BATCH), jnp.float32),
    )(g3)


def kernel(emb_table, W, b, batch_w, batch_x, batch_w_lengths, batch_x_lengths):
    table_t = emb_table.T
    tp = _project_table(table_t, W, b)
    tp_lin = tp.reshape(2 * _SPLIT, _EMB)

    bx_t = batch_x.T.astype(jnp.int32)
    idxv = jnp.where(bx_t < _SPLIT, 2 * bx_t, 2 * (bx_t - _SPLIT) + 1)
    idx3 = (idxv.reshape(_SEQ, 2, 8, 64)
            .transpose(0, 2, 1, 3)
            .reshape(_NW, _NCH, _CH))

    if _PHASE_CUT == 1:
        return tp
    g = _sc_gather(tp_lin, idx3)
    if _PHASE_CUT == 2:
        return g
    g3 = g.reshape(_SEQ, 512, 128)
    out_t = _final_transpose(g3)
    return out_t.transpose(2, 0, 1)


# speedup vs baseline: 3.1312x; 1.0222x over previous
import functools

import jax
import jax.numpy as jnp
from jax import lax
from jax.experimental import pallas as pl
from jax.experimental.pallas import tpu as pltpu
from jax.experimental.pallas import tpu_sc as plsc

_VOCAB = 1000000
_SPLIT = 507904
_EMB = 64
_TAG = 64
_BATCH = 1024
_SEQ = 200

_NTOK = _BATCH * _SEQ
_NW = 32
_PER_W = _NTOK // _NW
_CH = 128
_NCH = _PER_W // _CH

_P1_BU = 16384
_P1_GRID = _SPLIT // _P1_BU
_PHASE_CUT = 0


def _p1_body(x1_ref, x2_ref, w_ref, b_ref, o_ref):
    w = w_ref[...]
    b = b_ref[...]
    dn = (((0,), (0,)), ((), ()))
    o_ref[:, 0:64] = (
        lax.dot_general(x1_ref[...], w, dn, preferred_element_type=jnp.float32)
        + b
    )
    o_ref[:, 64:128] = (
        lax.dot_general(x2_ref[...], w, dn, preferred_element_type=jnp.float32)
        + b
    )


def _project_table(table_t, W, b):
    return pl.pallas_call(
        _p1_body,
        grid=(_P1_GRID,),
        in_specs=[
            pl.BlockSpec((_EMB, _P1_BU), lambda i: (0, i)),
            pl.BlockSpec(
                (_EMB, _P1_BU),
                lambda i: (0, jnp.minimum(i + _P1_GRID, _VOCAB // _P1_BU)),
            ),
            pl.BlockSpec((_EMB, _TAG), lambda i: (0, 0)),
            pl.BlockSpec((1, _TAG), lambda i: (0, 0)),
        ],
        out_specs=pl.BlockSpec((_P1_BU, 2 * _TAG), lambda i: (i, 0)),
        out_shape=jax.ShapeDtypeStruct((_SPLIT, 2 * _TAG), jnp.float32),
    )(table_t, table_t, W, b.reshape(1, _TAG))


def _make_sc_gather():
    mesh = plsc.VectorSubcoreMesh(core_axis_name="c", subcore_axis_name="s")

    nsub = 5
    sup_rows = nsub * _CH
    nsup = _PER_W // sup_rows

    @functools.partial(
        pl.kernel,
        mesh=mesh,
        out_type=jax.ShapeDtypeStruct((_NTOK // 2, 2 * _EMB), jnp.float32),
        scratch_types=[
            pltpu.VMEM((_NCH, _CH), jnp.int32),
            pltpu.VMEM((sup_rows, _EMB), jnp.float32),
            pltpu.SemaphoreType.DMA,
            pltpu.SemaphoreType.DMA,
        ],
        compiler_params=pltpu.CompilerParams(use_tc_tiling_on_sc=False),
    )
    def gather_kernel(table_hbm, idx_hbm, out_hbm, idx_v, rows_v, gsem, wsem):
        wid = lax.axis_index("s") * 2 + lax.axis_index("c")
        rbase = wid * (_PER_W // 2)
        pltpu.sync_copy(idx_hbm.at[wid], idx_v)

        def body(g, carry):
            handles = []
            for j in range(nsub):
                h = pltpu.async_copy(
                    table_hbm.at[idx_v.at[g * nsub + j]],
                    rows_v.at[pl.ds(j * _CH, _CH)],
                    gsem,
                )
                handles.append(h)
            for h in handles:
                h.wait()
            ws = []
            for j in range(nsub):
                r0 = rbase + g * (sup_rows // 2) + j * 64
                ws.append(pltpu.async_copy(
                    rows_v.at[pl.ds(j * _CH, 64)],
                    out_hbm.at[pl.ds(r0, 64), pl.ds(0, 64)],
                    wsem,
                ))
                ws.append(pltpu.async_copy(
                    rows_v.at[pl.ds(j * _CH + 64, 64)],
                    out_hbm.at[pl.ds(r0, 64), pl.ds(64, 64)],
                    wsem,
                ))
            for h in ws:
                h.wait()
            return carry

        lax.fori_loop(0, nsup, body, 0)

    return gather_kernel


_sc_gather = _make_sc_gather()


_P3_SB = 20


def _p3_body(x_ref, o_ref):
    for si in range(_P3_SB):
        x = x_ref[si]
        o_ref[si, :, 0:512] = x[:, 0:64].T
        o_ref[si, :, 512:1024] = x[:, 64:128].T


def _final_transpose(g3):
    return pl.pallas_call(
        _p3_body,
        grid=(_SEQ // _P3_SB,),
        in_specs=[pl.BlockSpec((_P3_SB, 512, 128), lambda i: (i, 0, 0))],
        out_specs=pl.BlockSpec((_P3_SB, _TAG, _BATCH), lambda i: (i, 0, 0)),
        out_shape=jax.ShapeDtypeStruct((_SEQ, _TAG, _BATCH), jnp.float32),
    )(g3)


def kernel(emb_table, W, b, batch_w, batch_x, batch_w_lengths, batch_x_lengths):
    table_t = emb_table.T
    tp = _project_table(table_t, W, b)
    tp_lin = tp.reshape(2 * _SPLIT, _EMB)

    bx_t = batch_x.T.astype(jnp.int32)
    idxv = jnp.where(bx_t < _SPLIT, 2 * bx_t, 2 * (bx_t - _SPLIT) + 1)
    idx3 = (idxv.reshape(_SEQ, 2, 8, 64)
            .transpose(0, 2, 1, 3)
            .reshape(_NW, _NCH, _CH))

    if _PHASE_CUT == 1:
        return tp
    g = _sc_gather(tp_lin, idx3)
    if _PHASE_CUT == 2:
        return g
    g3 = g.reshape(_SEQ, 512, 128)
    out_t = _final_transpose(g3)
    return out_t.transpose(2, 0, 1)


# speedup vs baseline: 3.1439x; 1.0041x over previous
import functools

import jax
import jax.numpy as jnp
from jax import lax
from jax.experimental import pallas as pl
from jax.experimental.pallas import tpu as pltpu
from jax.experimental.pallas import tpu_sc as plsc

_VOCAB = 1000000
_SPLIT = 507904
_EMB = 64
_TAG = 64
_BATCH = 1024
_SEQ = 200

_NTOK = _BATCH * _SEQ
_NW = 32
_PER_W = _NTOK // _NW
_CH = 128
_NCH = _PER_W // _CH

_P1_BU = 16384
_P1_GRID = _SPLIT // _P1_BU


def _p1_body(x1_ref, x2_ref, w_ref, b_ref, o_ref):
    w = w_ref[...]
    b = b_ref[...]
    dn = (((0,), (0,)), ((), ()))
    o_ref[:, 0:64] = (
        lax.dot_general(x1_ref[...], w, dn, preferred_element_type=jnp.float32)
        + b
    )
    o_ref[:, 64:128] = (
        lax.dot_general(x2_ref[...], w, dn, preferred_element_type=jnp.float32)
        + b
    )


def _project_table(table_t, W, b):
    return pl.pallas_call(
        _p1_body,
        grid=(_P1_GRID,),
        in_specs=[
            pl.BlockSpec((_EMB, _P1_BU), lambda i: (0, i)),
            pl.BlockSpec(
                (_EMB, _P1_BU),
                lambda i: (0, jnp.minimum(i + _P1_GRID, _VOCAB // _P1_BU)),
            ),
            pl.BlockSpec((_EMB, _TAG), lambda i: (0, 0)),
            pl.BlockSpec((1, _TAG), lambda i: (0, 0)),
        ],
        out_specs=pl.BlockSpec((_P1_BU, 2 * _TAG), lambda i: (i, 0)),
        out_shape=jax.ShapeDtypeStruct((_SPLIT, 2 * _TAG), jnp.float32),
    )(table_t, table_t, W, b.reshape(1, _TAG))


def _make_sc_gather():
    mesh = plsc.VectorSubcoreMesh(core_axis_name="c", subcore_axis_name="s")

    nsub = 5
    sup_rows = nsub * _CH
    nsup = _PER_W // sup_rows

    @functools.partial(
        pl.kernel,
        mesh=mesh,
        out_type=jax.ShapeDtypeStruct((_NTOK // 2, 2 * _EMB), jnp.float32),
        scratch_types=[
            pltpu.VMEM((_NCH, _CH), jnp.int32),
            pltpu.VMEM((sup_rows, _EMB), jnp.float32),
            pltpu.SemaphoreType.DMA,
            pltpu.SemaphoreType.DMA,
        ],
        compiler_params=pltpu.CompilerParams(use_tc_tiling_on_sc=False),
    )
    def gather_kernel(table_hbm, idx_hbm, out_hbm, idx_v, rows_v, gsem, wsem):
        wid = lax.axis_index("s") * 2 + lax.axis_index("c")
        rbase = wid * (_PER_W // 2)
        pltpu.sync_copy(idx_hbm.at[wid], idx_v)

        def body(g, carry):
            handles = []
            for j in range(nsub):
                h = pltpu.async_copy(
                    table_hbm.at[idx_v.at[g * nsub + j]],
                    rows_v.at[pl.ds(j * _CH, _CH)],
                    gsem,
                )
                handles.append(h)
            for h in handles:
                h.wait()
            ws = []
            for j in range(nsub):
                r0 = rbase + g * (sup_rows // 2) + j * 64
                ws.append(pltpu.async_copy(
                    rows_v.at[pl.ds(j * _CH, 64)],
                    out_hbm.at[pl.ds(r0, 64), pl.ds(0, 64)],
                    wsem,
                ))
                ws.append(pltpu.async_copy(
                    rows_v.at[pl.ds(j * _CH + 64, 64)],
                    out_hbm.at[pl.ds(r0, 64), pl.ds(64, 64)],
                    wsem,
                ))
            for h in ws:
                h.wait()
            return carry

        lax.fori_loop(0, nsup, body, 0)

    return gather_kernel


_sc_gather = _make_sc_gather()


_P3_SB = 20


def _p3_body(x_ref, o_ref):
    for si in range(_P3_SB):
        x = x_ref[si]
        o_ref[si, :, 0:512] = x[:, 0:64].T
        o_ref[si, :, 512:1024] = x[:, 64:128].T


def _final_transpose(g3):
    return pl.pallas_call(
        _p3_body,
        grid=(_SEQ // _P3_SB,),
        in_specs=[pl.BlockSpec((_P3_SB, 512, 128), lambda i: (i, 0, 0))],
        out_specs=pl.BlockSpec((_P3_SB, _TAG, _BATCH), lambda i: (i, 0, 0)),
        out_shape=jax.ShapeDtypeStruct((_SEQ, _TAG, _BATCH), jnp.float32),
    )(g3)


def kernel(emb_table, W, b, batch_w, batch_x, batch_w_lengths, batch_x_lengths):
    table_t = emb_table.T
    tp = _project_table(table_t, W, b)
    tp_lin = tp.reshape(2 * _SPLIT, _EMB)

    bx_t = batch_x.T.astype(jnp.int32)
    idxv = jnp.where(bx_t < _SPLIT, 2 * bx_t, 2 * (bx_t - _SPLIT) + 1)
    idx3 = (idxv.reshape(_SEQ, 2, 8, 64)
            .transpose(0, 2, 1, 3)
            .reshape(_NW, _NCH, _CH))

    g = _sc_gather(tp_lin, idx3)
    g3 = g.reshape(_SEQ, 512, 128)
    out_t = _final_transpose(g3)
    return out_t.transpose(2, 0, 1)
